# Initial kernel scaffold; baseline (speedup 1.0000x reference)
import jax
import jax.numpy as jnp
from jax.experimental import pallas as pl


def kernel(items, items_aug, adjs, edge_msk, aug_adjs, edge_msk_aug1, msk, msk_aug, emb_table, W_gat, a_gat, Ws, bs, qs, W1, b1, W2, b2):
    raise NotImplementedError("write your pallas kernel here")


# speedup vs baseline: 1.5640x; 1.5640x over previous
import functools

import jax
import jax.numpy as jnp
from jax import lax
from jax.experimental import pallas as pl
from jax.experimental.pallas import tpu as pltpu
from jax.experimental.pallas import tpu_sc as plsc

NFEAT = 256
NHID = 64
SHID = 64
ALPHA = 0.2
NHEADS = 8
MP = 2
B = 128
N = 128
DH = NHID * NHEADS


_NIDX = 2 * B * N
_NC = 2
_NS = 16
_NW = _NC * _NS
_ROWS_PER_W = _NIDX // _NW
_CHUNK = 128
_NCHUNK = _ROWS_PER_W // _CHUNK


def _sc_gather_body(table_hbm, idx_hbm, out_hbm, idx_v, rows_v, sems):
    wid = lax.axis_index("s") * _NC + lax.axis_index("c")
    base = wid * _ROWS_PER_W
    pltpu.sync_copy(idx_hbm.at[pl.ds(base, _ROWS_PER_W)], idx_v)
    copies = [None, None]
    copies[0] = pltpu.async_copy(
        table_hbm.at[idx_v.at[pl.ds(0, _CHUNK)]], rows_v.at[0], sems.at[0])
    for c in range(_NCHUNK):
        cur = c % 2
        nxt = (c + 1) % 2
        if c + 1 < _NCHUNK:
            copies[nxt] = pltpu.async_copy(
                table_hbm.at[idx_v.at[pl.ds((c + 1) * _CHUNK, _CHUNK)]],
                rows_v.at[nxt], sems.at[nxt])
        copies[cur].wait()
        pltpu.sync_copy(rows_v.at[cur],
                        out_hbm.at[pl.ds(base + c * _CHUNK, _CHUNK)])


def _sc_gather(table, idx):
    mesh = plsc.VectorSubcoreMesh(core_axis_name="c", subcore_axis_name="s")
    k = pl.kernel(
        _sc_gather_body,
        out_type=jax.ShapeDtypeStruct((_NIDX, NFEAT), jnp.float32),
        mesh=mesh,
        scratch_types=[
            pltpu.VMEM((_ROWS_PER_W,), jnp.int32),
            pltpu.VMEM((2, _CHUNK, NFEAT), jnp.float32),
            pltpu.SemaphoreType.DMA((2,)),
        ],
    )
    return k(table, idx)


def _enc_body(seq_ref, adj_ref, emsk_ref, msk_ref, wr_ref, a1_ref, a2t_ref,
              ws_ref, bs_ref, qs_ref, out_ref):
    i = pl.program_id(0)
    seq = seq_ref[0]
    msk_row = msk_ref[pl.ds(i, 1), :]
    zs = []
    wsem = []
    for mp in range(MP):
        wr = wr_ref[mp]
        wh = jnp.dot(seq, wr, preferred_element_type=jnp.float32)
        e1 = jnp.dot(wh, a1_ref[mp], preferred_element_type=jnp.float32)
        e2t = lax.dot_general(a2t_ref[mp], wh, (((1,), (1,)), ((), ())),
                              preferred_element_type=jnp.float32)
        mask = (adj_ref[mp, 0] * emsk_ref[mp, 0]) > 0.5
        outs = []
        for h in range(NHEADS):
            e = e1[:, h:h + 1] + e2t[h:h + 1, :]
            e = jnp.maximum(e, ALPHA * e)
            e = jnp.where(mask, e, -1e9)
            m = jnp.max(e, axis=1, keepdims=True)
            p = jnp.exp(e - m)
            s = jnp.sum(p, axis=1, keepdims=True)
            attn = p / s
            outs.append(jnp.dot(attn, wh[:, h * NHID:(h + 1) * NHID],
                                preferred_element_type=jnp.float32))
        z = jnp.concatenate(outs, axis=1)
        z = jnp.where(z > 0, z, jnp.exp(jnp.minimum(z, 0.0)) - 1.0)
        zs.append(z)
        t = jnp.tanh(jnp.dot(z, ws_ref[...], preferred_element_type=jnp.float32)
                     + bs_ref[...])
        wv = jnp.dot(t, qs_ref[...], preferred_element_type=jnp.float32)
        wsem.append(jnp.sum(wv, axis=0, keepdims=True) * (1.0 / N))
    m = jnp.maximum(wsem[0], wsem[1])
    b0 = jnp.exp(wsem[0] - m)
    b1 = jnp.exp(wsem[1] - m)
    z = (b0 * zs[0] + b1 * zs[1]) / (b0 + b1)
    denom = jnp.sum(msk_row, axis=1, keepdims=True) + 1e-8
    h = jnp.dot(msk_row, z, preferred_element_type=jnp.float32) / denom
    out_ref[...] = h.reshape(1, 1, DH)


def _encode(seq, adj, emsk, msk, w_r, a1m, a2t, ws, bs2, qs2):
    return pl.pallas_call(
        _enc_body,
        grid=(B,),
        in_specs=[
            pl.BlockSpec((1, N, NFEAT), lambda i: (i, 0, 0)),
            pl.BlockSpec((MP, 1, N, N), lambda i: (0, i, 0, 0)),
            pl.BlockSpec((MP, 1, N, N), lambda i: (0, i, 0, 0)),
            pl.BlockSpec((B, N), lambda i: (0, 0)),
            pl.BlockSpec((MP, NFEAT, DH), lambda i: (0, 0, 0)),
            pl.BlockSpec((MP, DH, NHEADS), lambda i: (0, 0, 0)),
            pl.BlockSpec((MP, NHEADS, DH), lambda i: (0, 0, 0)),
            pl.BlockSpec((DH, SHID), lambda i: (0, 0)),
            pl.BlockSpec((1, SHID), lambda i: (0, 0)),
            pl.BlockSpec((SHID, 1), lambda i: (0, 0)),
        ],
        out_specs=pl.BlockSpec((1, 1, DH), lambda i: (i, 0, 0)),
        out_shape=jax.ShapeDtypeStruct((B, 1, DH), jnp.float32),
        compiler_params=pltpu.CompilerParams(
            dimension_semantics=("arbitrary",)),
    )(seq, adj, emsk, msk, w_r, a1m, a2t, ws, bs2, qs2).reshape(B, DH)


def _proj_body(h_ref, w1_ref, b1_ref, w2_ref, b2_ref, out_ref):
    x = jnp.dot(h_ref[...], w1_ref[...], preferred_element_type=jnp.float32)
    x = jnp.maximum(x + b1_ref[...], 0.0)
    out_ref[...] = (jnp.dot(x, w2_ref[...], preferred_element_type=jnp.float32)
                    + b2_ref[...])


def _proj(hcat, w1, b1_2, w2, b2_2):
    return pl.pallas_call(
        _proj_body,
        out_shape=jax.ShapeDtypeStruct((2 * B, DH), jnp.float32),
    )(hcat, w1, b1_2, w2, b2_2)


def kernel(items, items_aug, adjs, edge_msk, aug_adjs, edge_msk_aug1, msk,
           msk_aug, emb_table, W_gat, a_gat, Ws, bs, qs, W1, b1, W2, b2):
    idx = jnp.concatenate(
        [items_aug.reshape(-1), items.reshape(-1)]).astype(jnp.int32)
    rows = _sc_gather(emb_table, idx)
    seq3 = rows[:B * N].reshape(B, N, NFEAT)
    seq1 = rows[B * N:].reshape(B, N, NFEAT)

    w_r = jnp.transpose(W_gat, (0, 2, 1, 3)).reshape(MP, NFEAT, DH)
    a1 = a_gat[:, :, :NHID]
    a2 = a_gat[:, :, NHID:]
    a1m = jnp.zeros((MP, DH, NHEADS), jnp.float32)
    a2t = jnp.zeros((MP, NHEADS, DH), jnp.float32)
    for h in range(NHEADS):
        a1m = a1m.at[:, h * NHID:(h + 1) * NHID, h].set(a1[:, h, :])
        a2t = a2t.at[:, h, h * NHID:(h + 1) * NHID].set(a2[:, h, :])
    bs2 = bs.reshape(1, SHID)
    qs2 = qs.reshape(SHID, 1)

    h_1 = _encode(seq3, aug_adjs, edge_msk_aug1, msk_aug, w_r, a1m, a2t,
                  Ws, bs2, qs2)
    h_0 = _encode(seq1, adjs, edge_msk, msk, w_r, a1m, a2t, Ws, bs2, qs2)

    hcat = jnp.concatenate([h_1, h_0], axis=0)
    p = _proj(hcat, W1, b1.reshape(1, DH), W2, b2.reshape(1, DH))
    return (p[:B], p[B:])


# speedup vs baseline: 1.7906x; 1.1449x over previous
import functools

import jax
import jax.numpy as jnp
from jax import lax
from jax.experimental import pallas as pl
from jax.experimental.pallas import tpu as pltpu
from jax.experimental.pallas import tpu_sc as plsc

NFEAT = 256
NHID = 64
SHID = 64
ALPHA = 0.2
NHEADS = 8
MP = 2
B = 128
N = 128
DH = NHID * NHEADS


_NIDX = 2 * B * N
_NC = 2
_NS = 16
_NW = _NC * _NS
_ROWS_PER_W = _NIDX // _NW
_CHUNK = 128
_NCHUNK = _ROWS_PER_W // _CHUNK


def _sc_gather_body(table_hbm, idx_hbm, out_hbm, idx_v, rows_v, sems):
    wid = lax.axis_index("s") * _NC + lax.axis_index("c")
    base = wid * _ROWS_PER_W
    pltpu.sync_copy(idx_hbm.at[pl.ds(base, _ROWS_PER_W)], idx_v)
    copies = [None, None]
    copies[0] = pltpu.async_copy(
        table_hbm.at[idx_v.at[pl.ds(0, _CHUNK)]], rows_v.at[0], sems.at[0])
    for c in range(_NCHUNK):
        cur = c % 2
        nxt = (c + 1) % 2
        if c + 1 < _NCHUNK:
            copies[nxt] = pltpu.async_copy(
                table_hbm.at[idx_v.at[pl.ds((c + 1) * _CHUNK, _CHUNK)]],
                rows_v.at[nxt], sems.at[nxt])
        copies[cur].wait()
        pltpu.sync_copy(rows_v.at[cur],
                        out_hbm.at[pl.ds(base + c * _CHUNK, _CHUNK)])


def _sc_gather(table, idx):
    mesh = plsc.VectorSubcoreMesh(core_axis_name="c", subcore_axis_name="s")
    k = pl.kernel(
        _sc_gather_body,
        out_type=jax.ShapeDtypeStruct((_NIDX, NFEAT), jnp.float32),
        mesh=mesh,
        scratch_types=[
            pltpu.VMEM((_ROWS_PER_W,), jnp.int32),
            pltpu.VMEM((2, _CHUNK, NFEAT), jnp.float32),
            pltpu.SemaphoreType.DMA((2,)),
        ],
    )
    return k(table, idx)


def _enc_body(seq_ref, adj_ref, emsk_ref, msk_ref, wr_ref, a1_ref, a2t_ref,
              ws_ref, bs_ref, qs_ref, out_ref):
    i = pl.program_id(0)
    seq = seq_ref[0]
    msk_row = msk_ref[pl.ds(i, 1), :]
    zs = []
    wsem = []
    for mp in range(MP):
        wr = wr_ref[mp]
        wh = jnp.dot(seq, wr, preferred_element_type=jnp.float32)
        e1 = jnp.dot(wh, a1_ref[mp], preferred_element_type=jnp.float32)
        e2t = lax.dot_general(a2t_ref[mp], wh, (((1,), (1,)), ((), ())),
                              preferred_element_type=jnp.float32)
        mask = (adj_ref[mp, 0] * emsk_ref[mp, 0]) > 0.5
        ones_col = jnp.ones((N, 1), jnp.float32)
        whmean = jnp.dot(jnp.ones((1, N), jnp.float32), wh,
                         preferred_element_type=jnp.float32) * (1.0 / N)
        outs = []
        for h in range(NHEADS):
            e = e1[:, h:h + 1] + e2t[h:h + 1, :]
            e = jnp.maximum(e, ALPHA * e)
            p = jnp.where(mask, jnp.exp(e), 0.0)
            s = jnp.dot(p, ones_col, preferred_element_type=jnp.float32)
            z0 = jnp.where(s > 0.0, 0.0, 1.0)
            inv = 1.0 / (s + z0)
            out_u = jnp.dot(p, wh[:, h * NHID:(h + 1) * NHID],
                            preferred_element_type=jnp.float32)
            outs.append(out_u * inv
                        + z0 * whmean[:, h * NHID:(h + 1) * NHID])
        z = jnp.concatenate(outs, axis=1)
        z = jnp.where(z > 0, z, jnp.exp(jnp.minimum(z, 0.0)) - 1.0)
        zs.append(z)
        t = jnp.tanh(jnp.dot(z, ws_ref[...], preferred_element_type=jnp.float32)
                     + bs_ref[...])
        wv = jnp.dot(t, qs_ref[...], preferred_element_type=jnp.float32)
        wsem.append(jnp.sum(wv, axis=0, keepdims=True) * (1.0 / N))
    m = jnp.maximum(wsem[0], wsem[1])
    b0 = jnp.exp(wsem[0] - m)
    b1 = jnp.exp(wsem[1] - m)
    z = (b0 * zs[0] + b1 * zs[1]) / (b0 + b1)
    denom = jnp.sum(msk_row, axis=1, keepdims=True) + 1e-8
    h = jnp.dot(msk_row, z, preferred_element_type=jnp.float32) / denom
    out_ref[...] = h.reshape(1, 1, DH)


def _encode(seq, adj, emsk, msk, w_r, a1m, a2t, ws, bs2, qs2):
    return pl.pallas_call(
        _enc_body,
        grid=(B,),
        in_specs=[
            pl.BlockSpec((1, N, NFEAT), lambda i: (i, 0, 0)),
            pl.BlockSpec((MP, 1, N, N), lambda i: (0, i, 0, 0)),
            pl.BlockSpec((MP, 1, N, N), lambda i: (0, i, 0, 0)),
            pl.BlockSpec((B, N), lambda i: (0, 0)),
            pl.BlockSpec((MP, NFEAT, DH), lambda i: (0, 0, 0)),
            pl.BlockSpec((MP, DH, NHEADS), lambda i: (0, 0, 0)),
            pl.BlockSpec((MP, NHEADS, DH), lambda i: (0, 0, 0)),
            pl.BlockSpec((DH, SHID), lambda i: (0, 0)),
            pl.BlockSpec((1, SHID), lambda i: (0, 0)),
            pl.BlockSpec((SHID, 1), lambda i: (0, 0)),
        ],
        out_specs=pl.BlockSpec((1, 1, DH), lambda i: (i, 0, 0)),
        out_shape=jax.ShapeDtypeStruct((B, 1, DH), jnp.float32),
        compiler_params=pltpu.CompilerParams(
            dimension_semantics=("arbitrary",)),
    )(seq, adj, emsk, msk, w_r, a1m, a2t, ws, bs2, qs2).reshape(B, DH)


def _proj_body(h_ref, w1_ref, b1_ref, w2_ref, b2_ref, out_ref):
    x = jnp.dot(h_ref[...], w1_ref[...], preferred_element_type=jnp.float32)
    x = jnp.maximum(x + b1_ref[...], 0.0)
    out_ref[...] = (jnp.dot(x, w2_ref[...], preferred_element_type=jnp.float32)
                    + b2_ref[...])


def _proj(hcat, w1, b1_2, w2, b2_2):
    return pl.pallas_call(
        _proj_body,
        out_shape=jax.ShapeDtypeStruct((2 * B, DH), jnp.float32),
    )(hcat, w1, b1_2, w2, b2_2)


def kernel(items, items_aug, adjs, edge_msk, aug_adjs, edge_msk_aug1, msk,
           msk_aug, emb_table, W_gat, a_gat, Ws, bs, qs, W1, b1, W2, b2):
    idx = jnp.concatenate(
        [items_aug.reshape(-1), items.reshape(-1)]).astype(jnp.int32)
    rows = _sc_gather(emb_table, idx)
    seq3 = rows[:B * N].reshape(B, N, NFEAT)
    seq1 = rows[B * N:].reshape(B, N, NFEAT)

    w_r = jnp.transpose(W_gat, (0, 2, 1, 3)).reshape(MP, NFEAT, DH)
    a1 = a_gat[:, :, :NHID]
    a2 = a_gat[:, :, NHID:]
    a1m = jnp.zeros((MP, DH, NHEADS), jnp.float32)
    a2t = jnp.zeros((MP, NHEADS, DH), jnp.float32)
    for h in range(NHEADS):
        a1m = a1m.at[:, h * NHID:(h + 1) * NHID, h].set(a1[:, h, :])
        a2t = a2t.at[:, h, h * NHID:(h + 1) * NHID].set(a2[:, h, :])
    bs2 = bs.reshape(1, SHID)
    qs2 = qs.reshape(SHID, 1)

    h_1 = _encode(seq3, aug_adjs, edge_msk_aug1, msk_aug, w_r, a1m, a2t,
                  Ws, bs2, qs2)
    h_0 = _encode(seq1, adjs, edge_msk, msk, w_r, a1m, a2t, Ws, bs2, qs2)

    hcat = jnp.concatenate([h_1, h_0], axis=0)
    p = _proj(hcat, W1, b1.reshape(1, DH), W2, b2.reshape(1, DH))
    return (p[:B], p[B:])


# speedup vs baseline: 2.6386x; 1.4736x over previous
import functools

import jax
import jax.numpy as jnp
from jax import lax
from jax.experimental import pallas as pl
from jax.experimental.pallas import tpu as pltpu
from jax.experimental.pallas import tpu_sc as plsc

NFEAT = 256
NHID = 64
SHID = 64
ALPHA = 0.2
NHEADS = 8
MP = 2
B = 128
N = 128
DH = NHID * NHEADS


_NIDX = 2 * B * N
_NC = 2
_NS = 16
_NW = _NC * _NS
_ROWS_PER_W = _NIDX // _NW
_CHUNK = 128
_NCHUNK = _ROWS_PER_W // _CHUNK


def _sc_gather_body(table_hbm, idx_hbm, out_hbm, idx_v, rows_v, sems):
    wid = lax.axis_index("s") * _NC + lax.axis_index("c")
    base = wid * _ROWS_PER_W
    pltpu.sync_copy(idx_hbm.at[pl.ds(base, _ROWS_PER_W)], idx_v)
    copies = [None, None]
    copies[0] = pltpu.async_copy(
        table_hbm.at[idx_v.at[pl.ds(0, _CHUNK)]], rows_v.at[0], sems.at[0])
    for c in range(_NCHUNK):
        cur = c % 2
        nxt = (c + 1) % 2
        if c + 1 < _NCHUNK:
            copies[nxt] = pltpu.async_copy(
                table_hbm.at[idx_v.at[pl.ds((c + 1) * _CHUNK, _CHUNK)]],
                rows_v.at[nxt], sems.at[nxt])
        copies[cur].wait()
        pltpu.sync_copy(rows_v.at[cur],
                        out_hbm.at[pl.ds(base + c * _CHUNK, _CHUNK)])


def _sc_gather(table, idx):
    mesh = plsc.VectorSubcoreMesh(core_axis_name="c", subcore_axis_name="s")
    k = pl.kernel(
        _sc_gather_body,
        out_type=jax.ShapeDtypeStruct((_NIDX, NFEAT), jnp.float32),
        mesh=mesh,
        scratch_types=[
            pltpu.VMEM((_ROWS_PER_W,), jnp.int32),
            pltpu.VMEM((2, _CHUNK, NFEAT), jnp.float32),
            pltpu.SemaphoreType.DMA((2,)),
        ],
    )
    return k(table, idx)


BB = 4


def _enc_body(seq_ref, adj_ref, emsk_ref, msk_ref, wr_ref, a1_ref, a2t_ref,
              ws_ref, bs_ref, qs_ref, out_ref):
    i = pl.program_id(0)
    seq_all = seq_ref[...].reshape(BB * N, NFEAT)
    ones_col = jnp.ones((N, 1), jnp.float32)
    wh_mp = []
    e1_mp = []
    e2t_mp = []
    for mp in range(MP):
        wh_all = jnp.dot(seq_all, wr_ref[mp],
                         preferred_element_type=jnp.float32)
        wh_mp.append(wh_all)
        e1_mp.append(jnp.dot(wh_all, a1_ref[mp],
                             preferred_element_type=jnp.float32))
        e2t_mp.append(lax.dot_general(
            a2t_ref[mp], wh_all, (((1,), (1,)), ((), ())),
            preferred_element_type=jnp.float32))
    for b in range(BB):
        msk_row = msk_ref[pl.ds(i * BB + b, 1), :]
        zs = []
        wsem = []
        for mp in range(MP):
            wh = wh_mp[mp][b * N:(b + 1) * N, :]
            e1 = e1_mp[mp][b * N:(b + 1) * N, :]
            e2t = e2t_mp[mp][:, b * N:(b + 1) * N]
            mask = (adj_ref[mp, b] * emsk_ref[mp, b]) > 0.5
            whmean = jnp.dot(jnp.ones((1, N), jnp.float32), wh,
                             preferred_element_type=jnp.float32) * (1.0 / N)
            outs = []
            for h in range(NHEADS):
                e = e1[:, h:h + 1] + e2t[h:h + 1, :]
                e = jnp.maximum(e, ALPHA * e)
                p = jnp.where(mask, jnp.exp(e), 0.0)
                s = jnp.dot(p, ones_col, preferred_element_type=jnp.float32)
                z0 = jnp.where(s > 0.0, 0.0, 1.0)
                inv = 1.0 / (s + z0)
                out_u = jnp.dot(p, wh[:, h * NHID:(h + 1) * NHID],
                                preferred_element_type=jnp.float32)
                outs.append(out_u * inv
                            + z0 * whmean[:, h * NHID:(h + 1) * NHID])
            z = jnp.concatenate(outs, axis=1)
            z = jnp.where(z > 0, z, jnp.exp(jnp.minimum(z, 0.0)) - 1.0)
            zs.append(z)
            t = jnp.tanh(jnp.dot(z, ws_ref[...],
                                 preferred_element_type=jnp.float32)
                         + bs_ref[...])
            wv = jnp.dot(t, qs_ref[...], preferred_element_type=jnp.float32)
            wsem.append(jnp.sum(wv, axis=0, keepdims=True) * (1.0 / N))
        m = jnp.maximum(wsem[0], wsem[1])
        b0 = jnp.exp(wsem[0] - m)
        b1 = jnp.exp(wsem[1] - m)
        z = (b0 * zs[0] + b1 * zs[1]) / (b0 + b1)
        denom = jnp.sum(msk_row, axis=1, keepdims=True) + 1e-8
        h = jnp.dot(msk_row, z, preferred_element_type=jnp.float32) / denom
        out_ref[b] = h


def _encode(seq, adj, emsk, msk, w_r, a1m, a2t, ws, bs2, qs2):
    return pl.pallas_call(
        _enc_body,
        grid=(B // BB,),
        in_specs=[
            pl.BlockSpec((BB, N, NFEAT), lambda i: (i, 0, 0)),
            pl.BlockSpec((MP, BB, N, N), lambda i: (0, i, 0, 0)),
            pl.BlockSpec((MP, BB, N, N), lambda i: (0, i, 0, 0)),
            pl.BlockSpec((B, N), lambda i: (0, 0)),
            pl.BlockSpec((MP, NFEAT, DH), lambda i: (0, 0, 0)),
            pl.BlockSpec((MP, DH, NHEADS), lambda i: (0, 0, 0)),
            pl.BlockSpec((MP, NHEADS, DH), lambda i: (0, 0, 0)),
            pl.BlockSpec((DH, SHID), lambda i: (0, 0)),
            pl.BlockSpec((1, SHID), lambda i: (0, 0)),
            pl.BlockSpec((SHID, 1), lambda i: (0, 0)),
        ],
        out_specs=pl.BlockSpec((BB, 1, DH), lambda i: (i, 0, 0)),
        out_shape=jax.ShapeDtypeStruct((B, 1, DH), jnp.float32),
        compiler_params=pltpu.CompilerParams(
            dimension_semantics=("arbitrary",)),
    )(seq, adj, emsk, msk, w_r, a1m, a2t, ws, bs2, qs2).reshape(B, DH)


def _proj_body(h_ref, w1_ref, b1_ref, w2_ref, b2_ref, out_ref):
    x = jnp.dot(h_ref[...], w1_ref[...], preferred_element_type=jnp.float32)
    x = jnp.maximum(x + b1_ref[...], 0.0)
    out_ref[...] = (jnp.dot(x, w2_ref[...], preferred_element_type=jnp.float32)
                    + b2_ref[...])


def _proj(hcat, w1, b1_2, w2, b2_2):
    return pl.pallas_call(
        _proj_body,
        out_shape=jax.ShapeDtypeStruct((2 * B, DH), jnp.float32),
    )(hcat, w1, b1_2, w2, b2_2)


def kernel(items, items_aug, adjs, edge_msk, aug_adjs, edge_msk_aug1, msk,
           msk_aug, emb_table, W_gat, a_gat, Ws, bs, qs, W1, b1, W2, b2):
    idx = jnp.concatenate(
        [items_aug.reshape(-1), items.reshape(-1)]).astype(jnp.int32)
    rows = _sc_gather(emb_table, idx)
    seq3 = rows[:B * N].reshape(B, N, NFEAT)
    seq1 = rows[B * N:].reshape(B, N, NFEAT)

    w_r = jnp.transpose(W_gat, (0, 2, 1, 3)).reshape(MP, NFEAT, DH)
    a1 = a_gat[:, :, :NHID]
    a2 = a_gat[:, :, NHID:]
    a1m = jnp.zeros((MP, DH, NHEADS), jnp.float32)
    a2t = jnp.zeros((MP, NHEADS, DH), jnp.float32)
    for h in range(NHEADS):
        a1m = a1m.at[:, h * NHID:(h + 1) * NHID, h].set(a1[:, h, :])
        a2t = a2t.at[:, h, h * NHID:(h + 1) * NHID].set(a2[:, h, :])
    bs2 = bs.reshape(1, SHID)
    qs2 = qs.reshape(SHID, 1)

    h_1 = _encode(seq3, aug_adjs, edge_msk_aug1, msk_aug, w_r, a1m, a2t,
                  Ws, bs2, qs2)
    h_0 = _encode(seq1, adjs, edge_msk, msk, w_r, a1m, a2t, Ws, bs2, qs2)

    hcat = jnp.concatenate([h_1, h_0], axis=0)
    p = _proj(hcat, W1, b1.reshape(1, DH), W2, b2.reshape(1, DH))
    return (p[:B], p[B:])


# speedup vs baseline: 3.1542x; 1.1954x over previous
import jax
import jax.numpy as jnp
from jax import lax
from jax.experimental import pallas as pl
from jax.experimental.pallas import tpu as pltpu
from jax.experimental.pallas import tpu_sc as plsc

NFEAT = 256
NHID = 64
SHID = 64
ALPHA = 0.2
NHEADS = 8
MP = 2
B = 128
N = 128
DH = NHID * NHEADS


_NIDX = B * N
_NC = 2
_NS = 16
_NW = _NC * _NS
_ROWS_PER_W = _NIDX // _NW
_CHUNK = 128
_NCHUNK = _ROWS_PER_W // _CHUNK


def _sc_gather_body(table_hbm, idx_hbm, out_hbm, idx_v, rows_v, sems):
    wid = lax.axis_index("s") * _NC + lax.axis_index("c")
    base = wid * _ROWS_PER_W
    pltpu.sync_copy(idx_hbm.at[pl.ds(base, _ROWS_PER_W)], idx_v)
    copies = [None, None]
    copies[0] = pltpu.async_copy(
        table_hbm.at[idx_v.at[pl.ds(0, _CHUNK)]], rows_v.at[0], sems.at[0])
    for c in range(_NCHUNK):
        cur = c % 2
        nxt = (c + 1) % 2
        if c + 1 < _NCHUNK:
            copies[nxt] = pltpu.async_copy(
                table_hbm.at[idx_v.at[pl.ds((c + 1) * _CHUNK, _CHUNK)]],
                rows_v.at[nxt], sems.at[nxt])
        copies[cur].wait()
        pltpu.sync_copy(rows_v.at[cur],
                        out_hbm.at[pl.ds(base + c * _CHUNK, _CHUNK)])


def _sc_gather(table, idx):
    mesh = plsc.VectorSubcoreMesh(core_axis_name="c", subcore_axis_name="s")
    k = pl.kernel(
        _sc_gather_body,
        out_type=jax.ShapeDtypeStruct((_NIDX, NFEAT), jnp.float32),
        mesh=mesh,
        scratch_types=[
            pltpu.VMEM((_ROWS_PER_W,), jnp.int32),
            pltpu.VMEM((2, _CHUNK, NFEAT), jnp.float32),
            pltpu.SemaphoreType.DMA((2,)),
        ],
    )
    return k(table, idx)


BB = 8


def _enc_body(seq_ref, adj_ref, emsk_ref, msk_ref, wr_ref, a1_ref, a2t_ref,
              ws_ref, bs_ref, qs_ref, out_ref):
    i = pl.program_id(0)
    seq_all = seq_ref[...].reshape(BB * N, NFEAT).astype(jnp.bfloat16)
    ones_col = jnp.ones((N, 1), jnp.bfloat16)
    wh_mp = []
    e1_mp = []
    e2t_mp = []
    for mp in range(MP):
        wh_all = jnp.dot(seq_all, wr_ref[mp],
                         preferred_element_type=jnp.float32)
        wh_bf = wh_all.astype(jnp.bfloat16)
        wh_mp.append(wh_bf)
        e1_mp.append(jnp.dot(wh_bf, a1_ref[mp],
                             preferred_element_type=jnp.float32))
        e2t_mp.append(lax.dot_general(
            a2t_ref[mp], wh_bf, (((1,), (1,)), ((), ())),
            preferred_element_type=jnp.float32))
    for b in range(BB):
        msk_row = msk_ref[pl.ds(i * BB + b, 1), :]
        zs = []
        wsem = []
        for mp in range(MP):
            wh = wh_mp[mp][b * N:(b + 1) * N, :]
            e1 = e1_mp[mp][b * N:(b + 1) * N, :]
            e2t = e2t_mp[mp][:, b * N:(b + 1) * N]
            mask = (adj_ref[mp, b] * emsk_ref[mp, b]) > 0.5
            whmean = (jnp.dot(jnp.ones((1, N), jnp.bfloat16), wh,
                              preferred_element_type=jnp.float32)
                      * (1.0 / N)).astype(jnp.bfloat16)
            outs = []
            for h in range(NHEADS):
                e = e1[:, h:h + 1] + e2t[h:h + 1, :]
                e = jnp.maximum(e, ALPHA * e)
                p = jnp.where(mask, jnp.exp(e), 0.0).astype(jnp.bfloat16)
                s = jnp.dot(p, ones_col, preferred_element_type=jnp.float32)
                z0 = jnp.where(s > 0.0, 0.0, 1.0)
                inv = (1.0 / (s + z0)).astype(jnp.bfloat16)
                out_u = jnp.dot(p, wh[:, h * NHID:(h + 1) * NHID],
                                preferred_element_type=jnp.float32
                                ).astype(jnp.bfloat16)
                outs.append(out_u * inv
                            + z0.astype(jnp.bfloat16)
                            * whmean[:, h * NHID:(h + 1) * NHID])
            z = jnp.concatenate(outs, axis=1)
            zneg = (jnp.exp(jnp.minimum(z.astype(jnp.float32), 0.0))
                    - 1.0).astype(jnp.bfloat16)
            z = jnp.where(z > 0, z, zneg)
            zs.append(z)
            t = jnp.tanh(jnp.dot(z, ws_ref[...],
                                 preferred_element_type=jnp.float32)
                         + bs_ref[...])
            wv = jnp.dot(t.astype(jnp.bfloat16), qs_ref[...],
                         preferred_element_type=jnp.float32)
            wsem.append(jnp.sum(wv, axis=0, keepdims=True) * (1.0 / N))
        m = jnp.maximum(wsem[0], wsem[1])
        b0 = jnp.exp(wsem[0] - m)
        b1 = jnp.exp(wsem[1] - m)
        zc = (b0.astype(jnp.bfloat16) * zs[0]
              + b1.astype(jnp.bfloat16) * zs[1]) / (b0 + b1).astype(jnp.bfloat16)
        denom = jnp.sum(msk_row, axis=1, keepdims=True) + 1e-8
        h = jnp.dot(msk_row.astype(jnp.bfloat16), zc,
                    preferred_element_type=jnp.float32) / denom
        out_ref[b] = h


def _encode(seq, adj, emsk, msk, w_r, a1m, a2t, ws, bs2, qs2):
    return pl.pallas_call(
        _enc_body,
        grid=(B // BB,),
        in_specs=[
            pl.BlockSpec((BB, N, NFEAT), lambda i: (i, 0, 0)),
            pl.BlockSpec((MP, BB, N, N), lambda i: (0, i, 0, 0)),
            pl.BlockSpec((MP, BB, N, N), lambda i: (0, i, 0, 0)),
            pl.BlockSpec((B, N), lambda i: (0, 0)),
            pl.BlockSpec((MP, NFEAT, DH), lambda i: (0, 0, 0)),
            pl.BlockSpec((MP, DH, NHEADS), lambda i: (0, 0, 0)),
            pl.BlockSpec((MP, NHEADS, DH), lambda i: (0, 0, 0)),
            pl.BlockSpec((DH, SHID), lambda i: (0, 0)),
            pl.BlockSpec((1, SHID), lambda i: (0, 0)),
            pl.BlockSpec((SHID, 1), lambda i: (0, 0)),
        ],
        out_specs=pl.BlockSpec((BB, 1, DH), lambda i: (i, 0, 0)),
        out_shape=jax.ShapeDtypeStruct((B, 1, DH), jnp.float32),
        compiler_params=pltpu.CompilerParams(
            dimension_semantics=("arbitrary",)),
    )(seq, adj, emsk, msk, w_r, a1m, a2t, ws, bs2, qs2).reshape(B, DH)


def _proj_body(h_ref, w1_ref, b1_ref, w2_ref, b2_ref, out_ref):
    x = jnp.dot(h_ref[...].astype(jnp.bfloat16), w1_ref[...],
                preferred_element_type=jnp.float32)
    x = jnp.maximum(x + b1_ref[...], 0.0)
    out_ref[...] = (jnp.dot(x.astype(jnp.bfloat16), w2_ref[...],
                            preferred_element_type=jnp.float32)
                    + b2_ref[...])


def _proj(hcat, w1, b1_2, w2, b2_2):
    return pl.pallas_call(
        _proj_body,
        out_shape=jax.ShapeDtypeStruct((2 * B, DH), jnp.float32),
    )(hcat, w1, b1_2, w2, b2_2)


def kernel(items, items_aug, adjs, edge_msk, aug_adjs, edge_msk_aug1, msk,
           msk_aug, emb_table, W_gat, a_gat, Ws, bs, qs, W1, b1, W2, b2):
    seq3 = _sc_gather(emb_table,
                      items_aug.reshape(-1).astype(jnp.int32)).reshape(
                          B, N, NFEAT)
    seq1 = _sc_gather(emb_table,
                      items.reshape(-1).astype(jnp.int32)).reshape(
                          B, N, NFEAT)

    w_r = jnp.transpose(W_gat, (0, 2, 1, 3)).reshape(MP, NFEAT, DH)
    a1 = a_gat[:, :, :NHID]
    a2 = a_gat[:, :, NHID:]
    a1m = jnp.zeros((MP, DH, NHEADS), jnp.float32)
    a2t = jnp.zeros((MP, NHEADS, DH), jnp.float32)
    for h in range(NHEADS):
        a1m = a1m.at[:, h * NHID:(h + 1) * NHID, h].set(a1[:, h, :])
        a2t = a2t.at[:, h, h * NHID:(h + 1) * NHID].set(a2[:, h, :])
    bs2 = bs.reshape(1, SHID)
    qs2 = qs.reshape(SHID, 1)

    w_r = w_r.astype(jnp.bfloat16)
    a1m = a1m.astype(jnp.bfloat16)
    a2t = a2t.astype(jnp.bfloat16)
    ws_bf = Ws.astype(jnp.bfloat16)
    qs2 = qs2.astype(jnp.bfloat16)

    h_1 = _encode(seq3, aug_adjs, edge_msk_aug1, msk_aug, w_r, a1m, a2t,
                  ws_bf, bs2, qs2)
    h_0 = _encode(seq1, adjs, edge_msk, msk, w_r, a1m, a2t, ws_bf, bs2, qs2)

    hcat = jnp.concatenate([h_1, h_0], axis=0)
    p = _proj(hcat, W1.astype(jnp.bfloat16), b1.reshape(1, DH),
              W2.astype(jnp.bfloat16), b2.reshape(1, DH))
    return (p[:B], p[B:])
